# baseline (speedup 1.0000x reference)
import functools

import jax
import jax.numpy as jnp
from jax import lax
from jax.experimental import pallas as pl
from jax.experimental.pallas import tpu as pltpu
from jax.experimental.pallas import tpu_sc as plsc

E = 8
H = 1024
Q = 512
T = 2048
BLK = 256
XS_ROWS = T * 2 + E * BLK
NBLK = XS_ROWS // BLK
NW = 32
TPW = T // NW
CH = 16


def _router_body(x_ref, gw_ref, d0_ref, d1_ref, wb0_ref, wb1_ref, bm_ref):
    logits = jnp.dot(x_ref[...], gw_ref[...].T,
                     preferred_element_type=jnp.float32)
    m = jnp.max(logits, axis=-1, keepdims=True)
    p = jnp.exp(logits - m)
    s = p / jnp.sum(p, axis=-1, keepdims=True)
    lane = lax.broadcasted_iota(jnp.int32, s.shape, 1)
    v1 = jnp.max(s, axis=-1, keepdims=True)
    i1 = jnp.min(jnp.where(s == v1, lane, E), axis=-1, keepdims=True)
    s2 = jnp.where(lane == i1, -jnp.inf, s)
    v2 = jnp.max(s2, axis=-1, keepdims=True)
    i2 = jnp.min(jnp.where(s2 == v2, lane, E), axis=-1, keepdims=True)
    denom = v1 + v2 + 1e-9
    w1 = v1 / denom
    w2 = v2 / denom
    d2 = w1 + w2 + 1e-9
    r1 = w1 / d2
    r2 = w2 / d2

    m0 = (lane == i1).astype(jnp.float32)
    m1 = (lane == i2).astype(jnp.float32)
    mask = m0 + m1
    tr = lax.broadcasted_iota(jnp.int32, (T, T), 0)
    tc = lax.broadcasted_iota(jnp.int32, (T, T), 1)
    ltri = (tc < tr).astype(jnp.float32)
    rank = jnp.dot(ltri, mask, preferred_element_type=jnp.float32)
    counts = jnp.sum(mask, axis=0, keepdims=True)
    ci = counts.astype(jnp.int32)
    pci = ((ci + (BLK - 1)) // BLK) * BLK
    er = lax.broadcasted_iota(jnp.int32, (E, E), 0)
    ec = lax.broadcasted_iota(jnp.int32, (E, E), 1)
    excl = (er < ec).astype(jnp.float32)
    po = jnp.dot(pci.astype(jnp.float32), excl,
                 preferred_element_type=jnp.float32)

    d0 = jnp.sum(m0 * (rank + po), axis=-1, keepdims=True)
    d1 = jnp.sum(m1 * (rank + po), axis=-1, keepdims=True)
    d0_ref[...] = d0.astype(jnp.int32).reshape(NW, TPW)
    d1_ref[...] = d1.astype(jnp.int32).reshape(NW, TPW)
    ones = jnp.ones((1, 128), jnp.float32)
    wb0_ref[...] = r1 * ones
    wb1_ref[...] = r2 * ones

    poi = po.astype(jnp.int32)
    total = jnp.sum(pci, axis=-1, keepdims=True)
    br = lax.broadcasted_iota(jnp.int32, (NBLK, E), 0) * BLK
    po_b = jnp.broadcast_to(poi, (NBLK, E))
    cnt = jnp.sum((po_b <= br).astype(jnp.int32), axis=-1, keepdims=True)
    active = br[:, :1] < jnp.broadcast_to(total, (NBLK, 1))
    bm_ref[...] = jnp.where(active, cnt - 1, E)


def _router(hidden, gate_w):
    return pl.pallas_call(
        _router_body,
        out_shape=(
            jax.ShapeDtypeStruct((NW, TPW), jnp.int32),
            jax.ShapeDtypeStruct((NW, TPW), jnp.int32),
            jax.ShapeDtypeStruct((T, 128), jnp.float32),
            jax.ShapeDtypeStruct((T, 128), jnp.float32),
            jax.ShapeDtypeStruct((NBLK, 1), jnp.int32),
        ),
    )(hidden, gate_w)


@functools.lru_cache(maxsize=None)
def _get_dispatch():
    mesh = plsc.VectorSubcoreMesh(core_axis_name="c", subcore_axis_name="s")

    @functools.partial(
        pl.kernel,
        out_type=(
            jax.ShapeDtypeStruct((XS_ROWS, H), jnp.float32),
            jax.ShapeDtypeStruct((XS_ROWS, 128), jnp.float32),
        ),
        mesh=mesh,
        scratch_types=[
            pltpu.VMEM((TPW, H), jnp.float32),
            pltpu.VMEM((TPW,), jnp.int32),
            pltpu.VMEM((TPW,), jnp.int32),
            pltpu.VMEM((TPW, 128), jnp.float32),
            pltpu.VMEM((TPW, 128), jnp.float32),
            pltpu.SemaphoreType.DMA,
            pltpu.SemaphoreType.DMA,
        ],
    )
    def _dispatch(x_hbm, dst0_hbm, dst1_hbm, wb0_hbm, wb1_hbm,
                  xs_hbm, ws_hbm, xbuf, idx0, idx1, wbuf0, wbuf1,
                  insem, outsem):
        wid = lax.axis_index("s") * 2 + lax.axis_index("c")
        base = wid * TPW
        cps = [
            pltpu.async_copy(x_hbm.at[pl.ds(base, TPW)], xbuf, insem),
            pltpu.async_copy(dst0_hbm.at[wid], idx0, insem),
            pltpu.async_copy(dst1_hbm.at[wid], idx1, insem),
            pltpu.async_copy(wb0_hbm.at[pl.ds(base, TPW)], wbuf0, insem),
            pltpu.async_copy(wb1_hbm.at[pl.ds(base, TPW)], wbuf1, insem),
        ]
        for cp in cps:
            cp.wait()
        sc = [
            pltpu.async_copy(xbuf, xs_hbm.at[idx0], outsem),
            pltpu.async_copy(xbuf, xs_hbm.at[idx1], outsem),
            pltpu.async_copy(wbuf0, ws_hbm.at[idx0], outsem),
            pltpu.async_copy(wbuf1, ws_hbm.at[idx1], outsem),
        ]
        for cp in sc:
            cp.wait()

    return _dispatch


def _gemm_body(bm_ref, xs_ref, ws_ref, gup_ref, dp_ref, out_ref):
    b = pl.program_id(0)

    @pl.when(bm_ref[b, 0] < E)
    def _():
        xb = xs_ref[...]
        gu = jnp.dot(xb, gup_ref[0], preferred_element_type=jnp.float32)
        gate = gu[:, :Q]
        up = gu[:, Q:]
        act = (gate * jax.nn.sigmoid(gate)) * up
        eo = jnp.dot(act, dp_ref[0], preferred_element_type=jnp.float32)
        out_ref[...] = eo * ws_ref[:, :1]


def _gemm(bm, xs, ws, gup, dp):
    grid_spec = pltpu.PrefetchScalarGridSpec(
        num_scalar_prefetch=1,
        grid=(NBLK,),
        in_specs=[
            pl.BlockSpec((BLK, H), lambda b, bm: (b, 0)),
            pl.BlockSpec((BLK, 128), lambda b, bm: (b, 0)),
            pl.BlockSpec((1, H, 2 * Q),
                         lambda b, bm: (jnp.minimum(bm[b, 0], E - 1), 0, 0)),
            pl.BlockSpec((1, Q, H),
                         lambda b, bm: (jnp.minimum(bm[b, 0], E - 1), 0, 0)),
        ],
        out_specs=pl.BlockSpec((BLK, H), lambda b, bm: (b, 0)),
    )
    return pl.pallas_call(
        _gemm_body,
        grid_spec=grid_spec,
        out_shape=jax.ShapeDtypeStruct((XS_ROWS, H), jnp.float32),
    )(bm, xs, ws, gup, dp)


@functools.lru_cache(maxsize=None)
def _get_combine():
    mesh = plsc.VectorSubcoreMesh(core_axis_name="c", subcore_axis_name="s")
    nch = TPW // CH

    @functools.partial(
        pl.kernel,
        out_type=jax.ShapeDtypeStruct((T, H), jnp.float32),
        mesh=mesh,
        scratch_types=[
            pltpu.VMEM((TPW,), jnp.int32),
            pltpu.VMEM((TPW,), jnp.int32),
            pltpu.VMEM((3, CH, H), jnp.float32),
            pltpu.VMEM((3, CH, H), jnp.float32),
            pltpu.SemaphoreType.DMA,
            pltpu.SemaphoreType.DMA,
        ],
    )
    def _combine(contrib_hbm, dst0_hbm, dst1_hbm, out_hbm,
                 idx0, idx1, bufs0, bufs1, gsem, wsem):
        wid = lax.axis_index("s") * 2 + lax.axis_index("c")
        base = wid * TPW
        pltpu.sync_copy(dst0_hbm.at[wid], idx0)
        pltpu.sync_copy(dst1_hbm.at[wid], idx1)

        writes = {}

        def issue(c):
            if c - 3 in writes:
                writes.pop(c - 3).wait()
            sl = pl.ds(c * CH, CH)
            return (
                pltpu.async_copy(contrib_hbm.at[idx0.at[sl]],
                                 bufs0.at[c % 3], gsem),
                pltpu.async_copy(contrib_hbm.at[idx1.at[sl]],
                                 bufs1.at[c % 3], gsem),
            )

        pend = {0: issue(0)}
        for c in range(nch):
            if c + 1 < nch:
                pend[c + 1] = issue(c + 1)
            for cp in pend.pop(c):
                cp.wait()
            b0 = bufs0.at[c % 3]
            b1 = bufs1.at[c % 3]

            def _row(i, _):
                for l in range(H // 16):
                    sl = pl.ds(l * 16, 16)
                    b0[i, sl] = b0[i, sl] + b1[i, sl]
                return 0

            lax.fori_loop(0, CH, _row, 0)
            writes[c] = pltpu.async_copy(
                bufs0.at[c % 3], out_hbm.at[pl.ds(base + c * CH, CH)], wsem)
        for wcp in writes.values():
            wcp.wait()

    return _combine


def _epilogue_body(in_ref, out_ref):
    out_ref[...] = in_ref[...]


def _epilogue(out):
    return pl.pallas_call(
        _epilogue_body,
        grid=(E,),
        in_specs=[pl.BlockSpec((T // E, H), lambda b: (b, 0))],
        out_specs=pl.BlockSpec((T // E, H), lambda b: (b, 0)),
        out_shape=jax.ShapeDtypeStruct((T, H), jnp.float32),
    )(out)


@jax.jit
def kernel(x, gate_w, gate_up_proj, down_proj):
    Bb, Tt, Hh = x.shape
    hidden = x.reshape(Tt, Hh)
    dst0, dst1, wb0, wb1, bm = _router(hidden, gate_w)
    xs, ws = _get_dispatch()(hidden, dst0, dst1, wb0, wb1)
    contrib = _gemm(bm, xs, ws, gate_up_proj, down_proj)
    out = _get_combine()(contrib, dst0, dst1)
    out = _epilogue(out)
    return out.reshape(Bb, Tt, Hh)


# speedup vs baseline: 1.1359x; 1.1359x over previous
import functools

import jax
import jax.numpy as jnp
from jax import lax
from jax.experimental import pallas as pl
from jax.experimental.pallas import tpu as pltpu
from jax.experimental.pallas import tpu_sc as plsc

E = 8
H = 1024
Q = 512
T = 2048
BLK = 256
XS_ROWS = T * 2 + E * BLK
NBLK = XS_ROWS // BLK
NW = 32
TPW = T // NW
CH = 16


def _router_body(x_ref, gw_ref, d0_ref, d1_ref, wb0_ref, wb1_ref, bm_ref):
    logits = jnp.dot(x_ref[...], gw_ref[...].T,
                     preferred_element_type=jnp.float32)
    m = jnp.max(logits, axis=-1, keepdims=True)
    p = jnp.exp(logits - m)
    s = p / jnp.sum(p, axis=-1, keepdims=True)
    lane = lax.broadcasted_iota(jnp.int32, s.shape, 1)
    v1 = jnp.max(s, axis=-1, keepdims=True)
    i1 = jnp.min(jnp.where(s == v1, lane, E), axis=-1, keepdims=True)
    s2 = jnp.where(lane == i1, -jnp.inf, s)
    v2 = jnp.max(s2, axis=-1, keepdims=True)
    i2 = jnp.min(jnp.where(s2 == v2, lane, E), axis=-1, keepdims=True)
    denom = v1 + v2 + 1e-9
    w1 = v1 / denom
    w2 = v2 / denom
    d2 = w1 + w2 + 1e-9
    r1 = w1 / d2
    r2 = w2 / d2

    m0 = (lane == i1).astype(jnp.float32)
    m1 = (lane == i2).astype(jnp.float32)
    mask = m0 + m1
    tr = lax.broadcasted_iota(jnp.int32, (T, T), 0)
    tc = lax.broadcasted_iota(jnp.int32, (T, T), 1)
    ltri = (tc < tr).astype(jnp.float32)
    rank = jnp.dot(ltri, mask, preferred_element_type=jnp.float32)
    counts = jnp.sum(mask, axis=0, keepdims=True)
    ci = counts.astype(jnp.int32)
    pci = ((ci + (BLK - 1)) // BLK) * BLK
    er = lax.broadcasted_iota(jnp.int32, (E, E), 0)
    ec = lax.broadcasted_iota(jnp.int32, (E, E), 1)
    excl = (er < ec).astype(jnp.float32)
    po = jnp.dot(pci.astype(jnp.float32), excl,
                 preferred_element_type=jnp.float32)

    d0 = jnp.sum(m0 * (rank + po), axis=-1, keepdims=True)
    d1 = jnp.sum(m1 * (rank + po), axis=-1, keepdims=True)
    d0_ref[...] = d0.astype(jnp.int32).reshape(NW, TPW)
    d1_ref[...] = d1.astype(jnp.int32).reshape(NW, TPW)
    ones = jnp.ones((1, 128), jnp.float32)
    wb0_ref[...] = r1 * ones
    wb1_ref[...] = r2 * ones

    poi = po.astype(jnp.int32)
    total = jnp.sum(pci, axis=-1, keepdims=True)
    br = lax.broadcasted_iota(jnp.int32, (NBLK, E), 0) * BLK
    po_b = jnp.broadcast_to(poi, (NBLK, E))
    cnt = jnp.sum((po_b <= br).astype(jnp.int32), axis=-1, keepdims=True)
    tot_b = jnp.broadcast_to(total, (NBLK, 1))
    active = br[:, :1] < tot_b
    expert_col = jnp.where(active, cnt - 1, E)
    biota = lax.broadcasted_iota(jnp.int32, (NBLK, 1), 0)
    row_col = jnp.minimum(biota, tot_b // BLK - 1)
    bm_ref[...] = jnp.concatenate([expert_col, row_col], axis=1)


def _router(hidden, gate_w):
    return pl.pallas_call(
        _router_body,
        out_shape=(
            jax.ShapeDtypeStruct((NW, TPW), jnp.int32),
            jax.ShapeDtypeStruct((NW, TPW), jnp.int32),
            jax.ShapeDtypeStruct((T, 128), jnp.float32),
            jax.ShapeDtypeStruct((T, 128), jnp.float32),
            jax.ShapeDtypeStruct((NBLK, 2), jnp.int32),
        ),
    )(hidden, gate_w)


@functools.lru_cache(maxsize=None)
def _get_dispatch():
    mesh = plsc.VectorSubcoreMesh(core_axis_name="c", subcore_axis_name="s")

    @functools.partial(
        pl.kernel,
        out_type=(
            jax.ShapeDtypeStruct((XS_ROWS, H), jnp.float32),
            jax.ShapeDtypeStruct((XS_ROWS, 128), jnp.float32),
        ),
        mesh=mesh,
        scratch_types=[
            pltpu.VMEM((TPW, H), jnp.float32),
            pltpu.VMEM((TPW,), jnp.int32),
            pltpu.VMEM((TPW,), jnp.int32),
            pltpu.VMEM((TPW, 128), jnp.float32),
            pltpu.VMEM((TPW, 128), jnp.float32),
            pltpu.SemaphoreType.DMA,
            pltpu.SemaphoreType.DMA,
        ],
    )
    def _dispatch(x_hbm, dst0_hbm, dst1_hbm, wb0_hbm, wb1_hbm,
                  xs_hbm, ws_hbm, xbuf, idx0, idx1, wbuf0, wbuf1,
                  insem, outsem):
        wid = lax.axis_index("s") * 2 + lax.axis_index("c")
        base = wid * TPW
        cps = [
            pltpu.async_copy(x_hbm.at[pl.ds(base, TPW)], xbuf, insem),
            pltpu.async_copy(dst0_hbm.at[wid], idx0, insem),
            pltpu.async_copy(dst1_hbm.at[wid], idx1, insem),
            pltpu.async_copy(wb0_hbm.at[pl.ds(base, TPW)], wbuf0, insem),
            pltpu.async_copy(wb1_hbm.at[pl.ds(base, TPW)], wbuf1, insem),
        ]
        for cp in cps:
            cp.wait()
        sc = [
            pltpu.async_copy(xbuf, xs_hbm.at[idx0], outsem),
            pltpu.async_copy(xbuf, xs_hbm.at[idx1], outsem),
            pltpu.async_copy(wbuf0, ws_hbm.at[idx0], outsem),
            pltpu.async_copy(wbuf1, ws_hbm.at[idx1], outsem),
        ]
        for cp in sc:
            cp.wait()

    return _dispatch


def _gemm_body(bm_ref, xs_ref, ws_ref, gup_ref, dp_ref, out_ref):
    b = pl.program_id(0)

    @pl.when(bm_ref[b, 0] < E)
    def _():
        xb = xs_ref[...]
        gu = jnp.dot(xb, gup_ref[0], preferred_element_type=jnp.float32)
        gate = gu[:, :Q]
        up = gu[:, Q:]
        act = (gate * jax.nn.sigmoid(gate)) * up
        eo = jnp.dot(act, dp_ref[0], preferred_element_type=jnp.float32)
        out_ref[...] = eo * ws_ref[:, :1]


def _gemm(bm, xs, ws, gup, dp):
    grid_spec = pltpu.PrefetchScalarGridSpec(
        num_scalar_prefetch=1,
        grid=(NBLK,),
        in_specs=[
            pl.BlockSpec((BLK, H), lambda b, bm: (bm[b, 1], 0)),
            pl.BlockSpec((BLK, 128), lambda b, bm: (bm[b, 1], 0)),
            pl.BlockSpec((1, H, 2 * Q),
                         lambda b, bm: (jnp.minimum(bm[b, 0], E - 1), 0, 0)),
            pl.BlockSpec((1, Q, H),
                         lambda b, bm: (jnp.minimum(bm[b, 0], E - 1), 0, 0)),
        ],
        out_specs=pl.BlockSpec((BLK, H), lambda b, bm: (bm[b, 1], 0)),
    )
    return pl.pallas_call(
        _gemm_body,
        grid_spec=grid_spec,
        out_shape=jax.ShapeDtypeStruct((XS_ROWS, H), jnp.float32),
    )(bm, xs, ws, gup, dp)


@functools.lru_cache(maxsize=None)
def _get_combine():
    mesh = plsc.VectorSubcoreMesh(core_axis_name="c", subcore_axis_name="s")
    nch = TPW // CH

    @functools.partial(
        pl.kernel,
        out_type=jax.ShapeDtypeStruct((T, H), jnp.float32),
        mesh=mesh,
        scratch_types=[
            pltpu.VMEM((TPW,), jnp.int32),
            pltpu.VMEM((TPW,), jnp.int32),
            pltpu.VMEM((3, CH, H), jnp.float32),
            pltpu.VMEM((3, CH, H), jnp.float32),
            pltpu.SemaphoreType.DMA,
            pltpu.SemaphoreType.DMA,
        ],
    )
    def _combine(contrib_hbm, dst0_hbm, dst1_hbm, out_hbm,
                 idx0, idx1, bufs0, bufs1, gsem, wsem):
        wid = lax.axis_index("s") * 2 + lax.axis_index("c")
        base = wid * TPW
        pltpu.sync_copy(dst0_hbm.at[wid], idx0)
        pltpu.sync_copy(dst1_hbm.at[wid], idx1)

        writes = {}

        def issue(c):
            if c - 3 in writes:
                writes.pop(c - 3).wait()
            sl = pl.ds(c * CH, CH)
            return (
                pltpu.async_copy(contrib_hbm.at[idx0.at[sl]],
                                 bufs0.at[c % 3], gsem),
                pltpu.async_copy(contrib_hbm.at[idx1.at[sl]],
                                 bufs1.at[c % 3], gsem),
            )

        pend = {0: issue(0)}
        for c in range(nch):
            if c + 1 < nch:
                pend[c + 1] = issue(c + 1)
            for cp in pend.pop(c):
                cp.wait()
            b0 = bufs0.at[c % 3]
            b1 = bufs1.at[c % 3]

            def _row(i, _):
                for l in range(H // 16):
                    sl = pl.ds(l * 16, 16)
                    b0[i, sl] = b0[i, sl] + b1[i, sl]
                return 0

            lax.fori_loop(0, CH, _row, 0)
            writes[c] = pltpu.async_copy(
                bufs0.at[c % 3], out_hbm.at[pl.ds(base + c * CH, CH)], wsem)
        for wcp in writes.values():
            wcp.wait()

    return _combine


@jax.jit
def kernel(x, gate_w, gate_up_proj, down_proj):
    Bb, Tt, Hh = x.shape
    hidden = x.reshape(Tt, Hh)
    dst0, dst1, wb0, wb1, bm = _router(hidden, gate_w)
    xs, ws = _get_dispatch()(hidden, dst0, dst1, wb0, wb1)
    contrib = _gemm(bm, xs, ws, gate_up_proj, down_proj)
    out = _get_combine()(contrib, dst0, dst1)
    return out.reshape(Bb, Tt, Hh)
